# baseline (speedup 1.0000x reference)
import functools

import numpy as np
import jax
import jax.numpy as jnp
from jax import lax
from jax.experimental import pallas as pl
from jax.experimental.pallas import tpu as pltpu
from jax.experimental.pallas import tpu_sc as plsc

_N = 512
_B = 128
_NCHUNK = 32
_RPC = _N // _NCHUNK
_NPAIR = 2
_BPW = 4

_tri = np.triu(np.ones((_N, _N), dtype=bool), k=0)
_rows_np, _cols_np = np.nonzero(_tri)
_M = _rows_np.size

_O = [k * _RPC * _N - (k * _RPC) * (k * _RPC - 1) // 2 for k in range(_NCHUNK)]
_O.append(_M)
_S = [_O[k + 1] - _O[k] for k in range(_NCHUNK)]
_PS = [(s + 15) // 16 * 16 for s in _S]
_TO = [0]
for _k in range(_NCHUNK):
    _TO.append(_TO[-1] + _PS[_k])
_PSMAX = max(_PS)

_tbl_np = np.zeros((_TO[-1],), dtype=np.int32)
for _k in range(_NCHUNK):
    _sl = slice(_O[_k], _O[_k + 1])
    _tbl_np[_TO[_k]:_TO[_k] + _S[_k]] = (
        ((_rows_np[_sl] - _k * _RPC) << 9) | _cols_np[_sl]).astype(np.int32)

assert all(o % 8 == 0 for o in _O) and all(t % 16 == 0 for t in _TO)


@functools.partial(
    pl.kernel,
    mesh=plsc.VectorSubcoreMesh(core_axis_name="c", subcore_axis_name="s"),
    compiler_params=pltpu.CompilerParams(needs_layout_passes=False),
    out_type=jax.ShapeDtypeStruct((_B * _M,), jnp.float32),
    scratch_types=[
        pltpu.VMEM((_NPAIR, _RPC, _N), jnp.float32),
        pltpu.VMEM((_NPAIR, _RPC, _N), jnp.float32),
        pltpu.VMEM((_PSMAX,), jnp.int32),
        pltpu.VMEM((_PSMAX,), jnp.int32),
        pltpu.VMEM((_PSMAX,), jnp.float32),
        pltpu.VMEM((_PSMAX,), jnp.float32),
        pltpu.VMEM((_PSMAX,), jnp.float32),
        pltpu.VMEM((_PSMAX,), jnp.float32),
        pltpu.SemaphoreType.DMA,
        pltpu.SemaphoreType.DMA,
        pltpu.SemaphoreType.DMA,
        pltpu.SemaphoreType.DMA,
        pltpu.SemaphoreType.DMA,
        pltpu.SemaphoreType.DMA,
    ],
)
def _triu_select(in_hbm, tbl_hbm, out_hbm,
                 in0, in1, tb0, tb1, oa0, ob0, oa1, ob1,
                 isem0, isem1, tsem0, tsem1, osem0, osem1):
    wid = lax.axis_index("s") * 2 + lax.axis_index("c")
    in_bufs, tbl_bufs = (in0, in1), (tb0, tb1)
    out_bufs = ((oa0, ob0), (oa1, ob1))
    in_sems, tbl_sems, out_sems = (isem0, isem1), (tsem0, tsem1), (osem0, osem1)

    units = [(k, p) for k in range(_NCHUNK) for p in range(_NPAIR)]

    def in_copy(u):
        k, p = units[u]
        b0 = wid * _BPW + p * 2
        return pltpu.make_async_copy(
            in_hbm.at[pl.ds(b0, 2), pl.ds(k * _RPC, _RPC), :],
            in_bufs[u % 2], in_sems[u % 2])

    def tbl_copy(k):
        return pltpu.make_async_copy(
            tbl_hbm.at[pl.ds(_TO[k], _PS[k])],
            tbl_bufs[k % 2].at[pl.ds(0, _PS[k])], tbl_sems[k % 2])

    def out_copies(u):
        k, p = units[u]
        b0 = wid * _BPW + p * 2
        return [
            pltpu.make_async_copy(
                out_bufs[u % 2][j].at[pl.ds(0, _S[k])],
                out_hbm.at[pl.ds((b0 + j) * _M + _O[k], _S[k])],
                out_sems[u % 2])
            for j in range(2)
        ]

    zero16 = jnp.zeros((16,), jnp.int32)
    one16 = jnp.ones((16,), jnp.int32)

    tbl_copy(0).start()
    in_copy(0).start()
    for u, (k, p) in enumerate(units):
        if p == 0 and k + 1 < _NCHUNK:
            tbl_copy(k + 1).start()
        if u + 1 < len(units):
            in_copy(u + 1).start()
        in_copy(u).wait()
        if p == 0:
            tbl_copy(k).wait()
        if u >= 2:
            for cp in out_copies(u - 2):
                cp.wait()

        in_buf, tbl_buf = in_bufs[u % 2], tbl_bufs[k % 2]
        outa, outb = out_bufs[u % 2]

        @plsc.parallel_loop(0, _PS[k], 16, unroll=4)
        def vec_body(j):
            idx = tbl_buf[pl.ds(j, 16)]
            r = lax.shift_right_logical(idx, 9)
            c = lax.bitwise_and(idx, _N - 1)
            outa[pl.ds(j, 16)] = plsc.load_gather(in_buf, [zero16, r, c])
            outb[pl.ds(j, 16)] = plsc.load_gather(in_buf, [one16, r, c])

        for cp in out_copies(u):
            cp.start()
    for cp in out_copies(len(units) - 2):
        cp.wait()
    for cp in out_copies(len(units) - 1):
        cp.wait()


def kernel(input):
    tbl = jnp.asarray(_tbl_np)
    out = _triu_select(input, tbl)
    return out.reshape(_B, _M)
